# baseline (speedup 1.0000x reference)
import jax
import jax.numpy as jnp
from jax import lax
from jax.experimental import pallas as pl
from jax.experimental.pallas import tpu as pltpu
from jax.experimental.pallas import tpu_sc as plsc

N = 19
L = 16
NEG = -1e30

OFF_X = 0
OFF_W1 = 32
OFF_AS1 = 96
OFF_AD1 = 160
OFF_B1 = 224
OFF_W2 = 288
OFF_AS2 = 8480
OFF_AD2 = 8608
OFF_B2 = 8736
OFF_WCR = 8864
OFF_WCM = 11296
OFF_WM = 11328
OFF_BM = 11360
OFF_SCAL = 11392
OFF_BC = 11408
TOT = 11424

S_AS2 = 0
S_AD2 = 32
S_ALPHA = 64
S_Y = 96


def _lrelu(t):
    return jnp.where(t >= 0.0, t, 0.2 * t)


def _body(x_hbm, mmse_hbm, w1_hbm, as1_hbm, ad1_hbm, b1_hbm, w2_hbm,
          as2_hbm, ad2_hbm, b2_hbm, wm_hbm, bm_hbm, wc_hbm, bc_hbm,
          out_hbm, buf, h_ref, u_ref, s_ref, ustage, out_v, shared_u, sem):
    tid = lax.axis_index("s")
    lane = lax.iota(jnp.int32, L)
    tail_mask = lane < (N - L)

    copies = [
        (x_hbm, pl.ds(OFF_X, N)),
        (mmse_hbm, pl.ds(OFF_SCAL, 1)),
        (w1_hbm, pl.ds(OFF_W1, 64)),
        (as1_hbm, pl.ds(OFF_AS1, 64)),
        (ad1_hbm, pl.ds(OFF_AD1, 64)),
        (b1_hbm, pl.ds(OFF_B1, 64)),
        (w2_hbm, pl.ds(OFF_W2, 8192)),
        (as2_hbm, pl.ds(OFF_AS2, 128)),
        (ad2_hbm, pl.ds(OFF_AD2, 128)),
        (b2_hbm, pl.ds(OFF_B2, 128)),
        (wm_hbm, pl.ds(OFF_WM, 32)),
        (bm_hbm, pl.ds(OFF_BM, 32)),
        (bc_hbm, pl.ds(OFF_BC, 1)),
    ]
    handles = [pltpu.async_copy(s, buf.at[d], sem) for s, d in copies]
    handles.append(
        pltpu.async_copy(wc_hbm.at[pl.ds(0, N * 128)],
                         buf.at[pl.ds(OFF_WCR, N * 128)], sem))
    handles.append(
        pltpu.async_copy(wc_hbm.at[pl.ds(N * 128, 32)],
                         buf.at[pl.ds(OFF_WCM, 32)], sem))
    for hnd in handles:
        hnd.wait()
    plsc.subcore_barrier()

    def vl(off):
        return buf[pl.ds(off, L)]

    def bcast(ref, idx):
        return plsc.load_gather(ref, [jnp.full((L,), idx, jnp.int32)])

    zero = jnp.zeros((L,), jnp.float32)
    lane0 = lane == 0

    def u_row(i):
        def u_step(c, uc):
            base = OFF_W2 + c * 64
            wc = bcast(buf, OFF_WCR + i * 128 + c)
            return tuple(uc[k] + wc * buf[pl.ds(base + 16 * k, L)]
                         for k in range(4))

        uk = lax.fori_loop(0, 128, u_step, (zero,) * 4, unroll=4)
        for k in range(4):
            ustage[pl.ds(16 * k, L)] = uk[k]
        pltpu.sync_copy(ustage, shared_u.at[pl.ds(i * 64, 64)])

    u_row(tid)

    @pl.when(tid < 3)
    def _second_row():
        u_row(tid + L)

    plsc.subcore_barrier()

    @pl.when(tid == 0)
    def _serial():
        pltpu.sync_copy(shared_u, u_ref)

        xa = vl(OFF_X)
        xb = jnp.where(tail_mask, vl(OFF_X + L), 0.0)

        acc_s = vl(OFF_W1) * vl(OFF_AS1)
        acc_d = vl(OFF_W1) * vl(OFF_AD1)
        for k in range(1, 4):
            acc_s = acc_s + vl(OFF_W1 + 16 * k) * vl(OFF_AS1 + 16 * k)
            acc_d = acc_d + vl(OFF_W1 + 16 * k) * vl(OFF_AD1 + 16 * k)
        cs1 = jnp.sum(acc_s)
        cd1 = jnp.sum(acc_d)

        as1a = xa * cs1
        as1b = xb * cs1

        w1k = [vl(OFF_W1 + 16 * k) for k in range(4)]
        b1k = [vl(OFF_B1 + 16 * k) for k in range(4)]

        def l1_step(i, carry):
            adi = bcast(buf, OFF_X + i) * cd1
            e_a = _lrelu(as1a + adi)
            e_b = jnp.where(tail_mask, _lrelu(as1b + adi), NEG)
            m = jnp.maximum(jnp.max(e_a), jnp.max(e_b))
            p_a = jnp.exp(e_a - m)
            p_b = jnp.exp(e_b - m)
            s = jnp.sum(p_a) + jnp.sum(p_b)
            num = jnp.sum(p_a * xa) + jnp.sum(p_b * xb)
            yv = jnp.broadcast_to(num, (L,)) / jnp.broadcast_to(s, (L,))
            for k in range(4):
                h_ref[pl.ds(i * 64 + 16 * k, L)] = jnp.maximum(
                    yv * w1k[k] + b1k[k], 0.0)
            return carry

        lax.fori_loop(0, N, l1_step, 0)

        def vsvd_step(c, carry):
            base = OFF_W2 + c * 64
            ss = bcast(buf, OFF_AS2 + c)
            sd = bcast(buf, OFF_AD2 + c)
            out = []
            for k in range(4):
                w = buf[pl.ds(base + 16 * k, L)]
                out.append(carry[k] + ss * w)
                out.append(carry[4 + k] + sd * w)
            return tuple(out[0::2]) + tuple(out[1::2])

        vsvd = lax.fori_loop(0, 128, vsvd_step, (zero,) * 8, unroll=2)
        vs2 = vsvd[:4]
        vd2 = vsvd[4:]

        s_ref[pl.ds(S_AS2 + L, L)] = jnp.full((L,), NEG, jnp.float32)

        def as2_step(i, carry):
            hk = [h_ref[pl.ds(i * 64 + 16 * k, L)] for k in range(4)]
            a_s = hk[0] * vs2[0]
            a_d = hk[0] * vd2[0]
            for k in range(1, 4):
                a_s = a_s + hk[k] * vs2[k]
                a_d = a_d + hk[k] * vd2[k]
            plsc.store_scatter(
                s_ref, [jnp.full((L,), S_AS2 + i, jnp.int32)],
                jnp.broadcast_to(jnp.sum(a_s), (L,)), mask=lane0)
            plsc.store_scatter(
                s_ref, [jnp.full((L,), S_AD2 + i, jnp.int32)],
                jnp.broadcast_to(jnp.sum(a_d), (L,)), mask=lane0)
            return carry

        lax.fori_loop(0, N, as2_step, 0)

        as2a = s_ref[pl.ds(S_AS2, L)]
        as2b = s_ref[pl.ds(S_AS2 + L, L)]

        def l2_step(i, p1v):
            ad2i = bcast(s_ref, S_AD2 + i)
            e_a = _lrelu(as2a + ad2i)
            e_b = _lrelu(as2b + ad2i)
            m = jnp.maximum(jnp.max(e_a), jnp.max(e_b))
            p_a = jnp.exp(e_a - m)
            p_b = jnp.exp(e_b - m)
            rs = 1.0 / jnp.broadcast_to(jnp.sum(p_a) + jnp.sum(p_b), (L,))
            s_ref[pl.ds(S_ALPHA, L)] = p_a
            s_ref[pl.ds(S_ALPHA + L, L)] = p_b

            def m_step(j, mc):
                pj = bcast(s_ref, S_ALPHA + j)
                return tuple(mc[k] + pj * h_ref[pl.ds(j * 64 + 16 * k, L)]
                             for k in range(4))

            mk = lax.fori_loop(0, N, m_step, (zero,) * 4)
            for k in range(4):
                p1v = p1v + mk[k] * rs * u_ref[pl.ds(i * 64 + 16 * k, L)]
            return p1v

        p1v = lax.fori_loop(0, N, l2_step, zero)

        def col_step(i, carry):
            return tuple(
                carry[k] + buf[pl.ds(OFF_WCR + i * 128 + 16 * k, L)]
                for k in range(8))

        cols = lax.fori_loop(0, N, col_step, (zero,) * 8)
        for k in range(8):
            p1v = p1v + cols[k] * vl(OFF_B2 + 16 * k)

        mmse = vl(OFF_SCAL)[0]
        for k in range(2):
            t = mmse * vl(OFF_WM + 16 * k) + vl(OFF_BM + 16 * k)
            p1v = p1v + t * vl(OFF_WCM + 16 * k)

        z = jnp.sum(p1v) + vl(OFF_BC)[0]
        zv = jnp.broadcast_to(z, (L,))
        out_v[...] = 1.0 / (1.0 + jnp.exp(-zv))
        pltpu.sync_copy(out_v, out_hbm)


@jax.jit
def _run(x, mmse, w1, as1, ad1, b1, w2, as2, ad2, b2, wm, bm, wc, bc):
    mesh = plsc.VectorSubcoreMesh(
        core_axis_name="c", subcore_axis_name="s", num_cores=1,
        num_subcores=16)
    f = pl.kernel(
        _body,
        out_type=jax.ShapeDtypeStruct((L,), jnp.float32),
        mesh=mesh,
        compiler_params=pltpu.CompilerParams(needs_layout_passes=False),
        scratch_types=[
            pltpu.VMEM((TOT,), jnp.float32),
            pltpu.VMEM((N * 64,), jnp.float32),
            pltpu.VMEM((N * 64 + 48,), jnp.float32),
            pltpu.VMEM((128,), jnp.float32),
            pltpu.VMEM((64,), jnp.float32),
            pltpu.VMEM((L,), jnp.float32),
            pltpu.VMEM_SHARED((N * 64 + 48,), jnp.float32),
            pltpu.SemaphoreType.DMA,
        ],
    )
    return f(x, mmse, w1, as1, ad1, b1, w2, as2, ad2, b2, wm, bm, wc, bc)


def kernel(eeg_dem_scores, mmse, W1, a_src1, a_dst1, b1, W2, a_src2,
           a_dst2, b2, Wm, bm, Wc, bc):
    out = _run(eeg_dem_scores.reshape(N), mmse, W1.reshape(64), a_src1,
               a_dst1, b1, W2.reshape(128 * 64), a_src2, a_dst2, b2,
               Wm.reshape(32), bm, Wc.reshape(2464), bc)
    return out[0:1].reshape(1, 1)


# speedup vs baseline: 1.0213x; 1.0213x over previous
import jax
import jax.numpy as jnp
from jax import lax
from jax.experimental import pallas as pl
from jax.experimental.pallas import tpu as pltpu
from jax.experimental.pallas import tpu_sc as plsc

N = 19
L = 16
NEG = -1e30

OFF_X = 0
OFF_W1 = 32
OFF_AS1 = 96
OFF_AD1 = 160
OFF_B1 = 224
OFF_W2 = 288
OFF_AS2 = 8480
OFF_AD2 = 8608
OFF_B2 = 8736
OFF_WCR = 8864
OFF_WCM = 11296
OFF_WM = 11328
OFF_BM = 11360
OFF_SCAL = 11392
OFF_BC = 11408
TOT = 11424

S_AS2 = 0
S_AD2 = 32
S_ALPHA = 64
S_Y = 96


def _lrelu(t):
    return jnp.where(t >= 0.0, t, 0.2 * t)


def _body(x_hbm, mmse_hbm, w1_hbm, as1_hbm, ad1_hbm, b1_hbm, w2_hbm,
          as2_hbm, ad2_hbm, b2_hbm, wm_hbm, bm_hbm, wc_hbm, bc_hbm,
          out_hbm, buf, h_ref, u_ref, s_ref, ustage, out_v, shared_u, sem):
    tid = lax.axis_index("s")
    lane = lax.iota(jnp.int32, L)
    tail_mask = lane < (N - L)

    copies = [
        (x_hbm, pl.ds(OFF_X, N)),
        (mmse_hbm, pl.ds(OFF_SCAL, 1)),
        (w1_hbm, pl.ds(OFF_W1, 64)),
        (as1_hbm, pl.ds(OFF_AS1, 64)),
        (ad1_hbm, pl.ds(OFF_AD1, 64)),
        (b1_hbm, pl.ds(OFF_B1, 64)),
        (w2_hbm, pl.ds(OFF_W2, 8192)),
        (as2_hbm, pl.ds(OFF_AS2, 128)),
        (ad2_hbm, pl.ds(OFF_AD2, 128)),
        (b2_hbm, pl.ds(OFF_B2, 128)),
        (wm_hbm, pl.ds(OFF_WM, 32)),
        (bm_hbm, pl.ds(OFF_BM, 32)),
        (bc_hbm, pl.ds(OFF_BC, 1)),
    ]
    handles = [pltpu.async_copy(s, buf.at[d], sem) for s, d in copies]
    handles.append(
        pltpu.async_copy(wc_hbm.at[pl.ds(0, N * 128)],
                         buf.at[pl.ds(OFF_WCR, N * 128)], sem))
    handles.append(
        pltpu.async_copy(wc_hbm.at[pl.ds(N * 128, 32)],
                         buf.at[pl.ds(OFF_WCM, 32)], sem))
    for hnd in handles:
        hnd.wait()
    plsc.subcore_barrier()

    def vl(off):
        return buf[pl.ds(off, L)]

    def bcast(ref, idx):
        return plsc.load_gather(ref, [jnp.full((L,), idx, jnp.int32)])

    zero = jnp.zeros((L,), jnp.float32)
    lane0 = lane == 0

    def u_row(i):
        def u_step(c, uc):
            base = OFF_W2 + c * 64
            wc = bcast(buf, OFF_WCR + i * 128 + c)
            return tuple(uc[k] + wc * buf[pl.ds(base + 16 * k, L)]
                         for k in range(4))

        uk = lax.fori_loop(0, 128, u_step, (zero,) * 4, unroll=4)
        for k in range(4):
            ustage[pl.ds(16 * k, L)] = uk[k]
        pltpu.sync_copy(ustage.at[pl.ds(0, 64)],
                        shared_u.at[pl.ds(i * 64, 64)])

    u_row(tid)

    @pl.when(tid < 3)
    def _second_row():
        u_row(tid + L)

    @pl.when(tid == 3)
    def _vsvd_tile():
        def vsvd_step(c, carry):
            base = OFF_W2 + c * 64
            ss = bcast(buf, OFF_AS2 + c)
            sd = bcast(buf, OFF_AD2 + c)
            out = []
            for k in range(4):
                w = buf[pl.ds(base + 16 * k, L)]
                out.append(carry[k] + ss * w)
                out.append(carry[4 + k] + sd * w)
            return tuple(out[0::2]) + tuple(out[1::2])

        vsvd = lax.fori_loop(0, 128, vsvd_step, (zero,) * 8, unroll=2)
        for k in range(8):
            ustage[pl.ds(16 * k, L)] = vsvd[k]
        pltpu.sync_copy(ustage, shared_u.at[pl.ds(1264, 128)])

    @pl.when(tid == 4)
    def _colsum_tile():
        def col_step(i, carry):
            return tuple(
                carry[k] + buf[pl.ds(OFF_WCR + i * 128 + 16 * k, L)]
                for k in range(8))

        cols = lax.fori_loop(0, N, col_step, (zero,) * 8)
        for k in range(8):
            ustage[pl.ds(16 * k, L)] = cols[k]
        pltpu.sync_copy(ustage, shared_u.at[pl.ds(1392, 128)])

    plsc.subcore_barrier()

    @pl.when(tid == 0)
    def _serial():
        pltpu.sync_copy(shared_u, u_ref)

        xa = vl(OFF_X)
        xb = jnp.where(tail_mask, vl(OFF_X + L), 0.0)

        acc_s = vl(OFF_W1) * vl(OFF_AS1)
        acc_d = vl(OFF_W1) * vl(OFF_AD1)
        for k in range(1, 4):
            acc_s = acc_s + vl(OFF_W1 + 16 * k) * vl(OFF_AS1 + 16 * k)
            acc_d = acc_d + vl(OFF_W1 + 16 * k) * vl(OFF_AD1 + 16 * k)
        cs1 = jnp.sum(acc_s)
        cd1 = jnp.sum(acc_d)

        as1a = xa * cs1
        as1b = xb * cs1

        w1k = [vl(OFF_W1 + 16 * k) for k in range(4)]
        b1k = [vl(OFF_B1 + 16 * k) for k in range(4)]

        def l1_step(i, carry):
            adi = bcast(buf, OFF_X + i) * cd1
            e_a = _lrelu(as1a + adi)
            e_b = jnp.where(tail_mask, _lrelu(as1b + adi), NEG)
            m = jnp.maximum(jnp.max(e_a), jnp.max(e_b))
            p_a = jnp.exp(e_a - m)
            p_b = jnp.exp(e_b - m)
            s = jnp.sum(p_a) + jnp.sum(p_b)
            num = jnp.sum(p_a * xa) + jnp.sum(p_b * xb)
            yv = jnp.broadcast_to(num, (L,)) / jnp.broadcast_to(s, (L,))
            for k in range(4):
                h_ref[pl.ds(i * 64 + 16 * k, L)] = jnp.maximum(
                    yv * w1k[k] + b1k[k], 0.0)
            return carry

        lax.fori_loop(0, N, l1_step, 0)

        vs2 = [u_ref[pl.ds(1264 + 16 * k, L)] for k in range(4)]
        vd2 = [u_ref[pl.ds(1328 + 16 * k, L)] for k in range(4)]

        s_ref[pl.ds(S_AS2 + L, L)] = jnp.full((L,), NEG, jnp.float32)

        def as2_step(i, carry):
            hk = [h_ref[pl.ds(i * 64 + 16 * k, L)] for k in range(4)]
            a_s = hk[0] * vs2[0]
            a_d = hk[0] * vd2[0]
            for k in range(1, 4):
                a_s = a_s + hk[k] * vs2[k]
                a_d = a_d + hk[k] * vd2[k]
            plsc.store_scatter(
                s_ref, [jnp.full((L,), S_AS2 + i, jnp.int32)],
                jnp.broadcast_to(jnp.sum(a_s), (L,)), mask=lane0)
            plsc.store_scatter(
                s_ref, [jnp.full((L,), S_AD2 + i, jnp.int32)],
                jnp.broadcast_to(jnp.sum(a_d), (L,)), mask=lane0)
            return carry

        lax.fori_loop(0, N, as2_step, 0)

        as2a = s_ref[pl.ds(S_AS2, L)]
        as2b = s_ref[pl.ds(S_AS2 + L, L)]

        def l2_step(i, p1v):
            ad2i = bcast(s_ref, S_AD2 + i)
            e_a = _lrelu(as2a + ad2i)
            e_b = _lrelu(as2b + ad2i)
            m = jnp.maximum(jnp.max(e_a), jnp.max(e_b))
            p_a = jnp.exp(e_a - m)
            p_b = jnp.exp(e_b - m)
            rs = 1.0 / jnp.broadcast_to(jnp.sum(p_a) + jnp.sum(p_b), (L,))
            s_ref[pl.ds(S_ALPHA, L)] = p_a
            s_ref[pl.ds(S_ALPHA + L, L)] = p_b

            def m_step(j, mc):
                pj = bcast(s_ref, S_ALPHA + j)
                return tuple(mc[k] + pj * h_ref[pl.ds(j * 64 + 16 * k, L)]
                             for k in range(4))

            mk = lax.fori_loop(0, N, m_step, (zero,) * 4, unroll=2)
            for k in range(4):
                p1v = p1v + mk[k] * rs * u_ref[pl.ds(i * 64 + 16 * k, L)]
            return p1v

        p1v = lax.fori_loop(0, N, l2_step, zero)

        for k in range(8):
            p1v = p1v + u_ref[pl.ds(1392 + 16 * k, L)] * vl(OFF_B2 + 16 * k)

        mmse = vl(OFF_SCAL)[0]
        for k in range(2):
            t = mmse * vl(OFF_WM + 16 * k) + vl(OFF_BM + 16 * k)
            p1v = p1v + t * vl(OFF_WCM + 16 * k)

        z = jnp.sum(p1v) + vl(OFF_BC)[0]
        zv = jnp.broadcast_to(z, (L,))
        out_v[...] = 1.0 / (1.0 + jnp.exp(-zv))
        pltpu.sync_copy(out_v, out_hbm)


@jax.jit
def _run(x, mmse, w1, as1, ad1, b1, w2, as2, ad2, b2, wm, bm, wc, bc):
    mesh = plsc.VectorSubcoreMesh(
        core_axis_name="c", subcore_axis_name="s", num_cores=1,
        num_subcores=16)
    f = pl.kernel(
        _body,
        out_type=jax.ShapeDtypeStruct((L,), jnp.float32),
        mesh=mesh,
        compiler_params=pltpu.CompilerParams(needs_layout_passes=False),
        scratch_types=[
            pltpu.VMEM((TOT,), jnp.float32),
            pltpu.VMEM((N * 64,), jnp.float32),
            pltpu.VMEM((1520,), jnp.float32),
            pltpu.VMEM((128,), jnp.float32),
            pltpu.VMEM((128,), jnp.float32),
            pltpu.VMEM((L,), jnp.float32),
            pltpu.VMEM_SHARED((1520,), jnp.float32),
            pltpu.SemaphoreType.DMA,
        ],
    )
    return f(x, mmse, w1, as1, ad1, b1, w2, as2, ad2, b2, wm, bm, wc, bc)


def kernel(eeg_dem_scores, mmse, W1, a_src1, a_dst1, b1, W2, a_src2,
           a_dst2, b2, Wm, bm, Wc, bc):
    out = _run(eeg_dem_scores.reshape(N), mmse, W1.reshape(64), a_src1,
               a_dst1, b1, W2.reshape(128 * 64), a_src2, a_dst2, b2,
               Wm.reshape(32), bm, Wc.reshape(2464), bc)
    return out[0:1].reshape(1, 1)


# speedup vs baseline: 1.0541x; 1.0320x over previous
import jax
import jax.numpy as jnp
from jax import lax
from jax.experimental import pallas as pl
from jax.experimental.pallas import tpu as pltpu
from jax.experimental.pallas import tpu_sc as plsc

N = 19
L = 16
NEG = -1e30

OFF_X = 0
OFF_W1 = 32
OFF_AS1 = 96
OFF_AD1 = 160
OFF_B1 = 224
OFF_W2 = 288
OFF_AS2 = 8480
OFF_AD2 = 8608
OFF_B2 = 8736
OFF_WCR = 8864
OFF_WCM = 11296
OFF_WM = 11328
OFF_BM = 11360
OFF_SCAL = 11392
OFF_BC = 11408
TOT = 11424

S_AS2 = 0
S_AD2 = 32
S_ALPHA = 64
S_Y = 96


def _lrelu(t):
    return jnp.where(t >= 0.0, t, 0.2 * t)


def _body(x_hbm, mmse_hbm, w1_hbm, as1_hbm, ad1_hbm, b1_hbm, w2_hbm,
          as2_hbm, ad2_hbm, b2_hbm, wm_hbm, bm_hbm, wc_hbm, bc_hbm,
          out_hbm, buf, h_ref, u_ref, s_ref, ustage, out_v, shared_u, sem):
    tid = lax.axis_index("s")
    lane = lax.iota(jnp.int32, L)
    tail_mask = lane < (N - L)

    copies = [
        (x_hbm, pl.ds(OFF_X, N)),
        (mmse_hbm, pl.ds(OFF_SCAL, 1)),
        (w1_hbm, pl.ds(OFF_W1, 64)),
        (as1_hbm, pl.ds(OFF_AS1, 64)),
        (ad1_hbm, pl.ds(OFF_AD1, 64)),
        (b1_hbm, pl.ds(OFF_B1, 64)),
        (w2_hbm, pl.ds(OFF_W2, 8192)),
        (as2_hbm, pl.ds(OFF_AS2, 128)),
        (ad2_hbm, pl.ds(OFF_AD2, 128)),
        (b2_hbm, pl.ds(OFF_B2, 128)),
        (wm_hbm, pl.ds(OFF_WM, 32)),
        (bm_hbm, pl.ds(OFF_BM, 32)),
        (bc_hbm, pl.ds(OFF_BC, 1)),
    ]
    handles = [pltpu.async_copy(s, buf.at[d], sem) for s, d in copies]
    handles.append(
        pltpu.async_copy(wc_hbm.at[pl.ds(0, N * 128)],
                         buf.at[pl.ds(OFF_WCR, N * 128)], sem))
    handles.append(
        pltpu.async_copy(wc_hbm.at[pl.ds(N * 128, 32)],
                         buf.at[pl.ds(OFF_WCM, 32)], sem))
    for hnd in handles:
        hnd.wait()
    plsc.subcore_barrier()

    def vl(off):
        return buf[pl.ds(off, L)]

    def bcast(ref, idx):
        return plsc.load_gather(ref, [jnp.full((L,), idx, jnp.int32)])

    zero = jnp.zeros((L,), jnp.float32)
    lane0 = lane == 0

    def u_row(i):
        def u_step(c, uc):
            base = OFF_W2 + c * 64
            wc = bcast(buf, OFF_WCR + i * 128 + c)
            return tuple(uc[k] + wc * buf[pl.ds(base + 16 * k, L)]
                         for k in range(4))

        uk = lax.fori_loop(0, 128, u_step, (zero,) * 4, unroll=4)
        for k in range(4):
            ustage[pl.ds(16 * k, L)] = uk[k]
        pltpu.sync_copy(ustage.at[pl.ds(0, 64)],
                        shared_u.at[pl.ds(i * 64, 64)])

    u_row(tid)

    @pl.when(tid < 3)
    def _second_row():
        u_row(tid + L)

    @pl.when(tid == 3)
    def _vsvd_tile():
        def vsvd_step(c, carry):
            base = OFF_W2 + c * 64
            ss = bcast(buf, OFF_AS2 + c)
            sd = bcast(buf, OFF_AD2 + c)
            out = []
            for k in range(4):
                w = buf[pl.ds(base + 16 * k, L)]
                out.append(carry[k] + ss * w)
                out.append(carry[4 + k] + sd * w)
            return tuple(out[0::2]) + tuple(out[1::2])

        vsvd = lax.fori_loop(0, 128, vsvd_step, (zero,) * 8, unroll=2)
        for k in range(8):
            ustage[pl.ds(16 * k, L)] = vsvd[k]
        pltpu.sync_copy(ustage, shared_u.at[pl.ds(1264, 128)])

    @pl.when(tid == 4)
    def _colsum_tile():
        def col_step(i, carry):
            return tuple(
                carry[k] + buf[pl.ds(OFF_WCR + i * 128 + 16 * k, L)]
                for k in range(8))

        cols = lax.fori_loop(0, N, col_step, (zero,) * 8)
        for k in range(8):
            ustage[pl.ds(16 * k, L)] = cols[k]
        pltpu.sync_copy(ustage, shared_u.at[pl.ds(1392, 128)])

    plsc.subcore_barrier()

    @pl.when(tid == 0)
    def _serial():
        pltpu.sync_copy(shared_u, u_ref)

        xa = vl(OFF_X)
        xb = jnp.where(tail_mask, vl(OFF_X + L), 0.0)

        acc_s = vl(OFF_W1) * vl(OFF_AS1)
        acc_d = vl(OFF_W1) * vl(OFF_AD1)
        for k in range(1, 4):
            acc_s = acc_s + vl(OFF_W1 + 16 * k) * vl(OFF_AS1 + 16 * k)
            acc_d = acc_d + vl(OFF_W1 + 16 * k) * vl(OFF_AD1 + 16 * k)
        cs1 = jnp.sum(acc_s)
        cd1 = jnp.sum(acc_d)

        ad1a = xa * cd1
        ad1b = xb * cd1
        negv = jnp.full((L,), NEG, jnp.float32)

        def l1max_step(j, carry):
            mxa, mxb = carry
            asj = bcast(buf, OFF_X + j) * cs1
            return (jnp.maximum(mxa, _lrelu(ad1a + asj)),
                    jnp.maximum(mxb, _lrelu(ad1b + asj)))

        mx1a, mx1b = lax.fori_loop(0, N, l1max_step, (negv, negv))

        def l1sum_step(j, carry):
            sa, sb, ya, yb = carry
            xj = bcast(buf, OFF_X + j)
            asj = xj * cs1
            p_a = jnp.exp(_lrelu(ad1a + asj) - mx1a)
            p_b = jnp.exp(_lrelu(ad1b + asj) - mx1b)
            return (sa + p_a, sb + p_b, ya + p_a * xj, yb + p_b * xj)

        sa, sb, ya, yb = lax.fori_loop(0, N, l1sum_step, (zero,) * 4)
        s_ref[pl.ds(S_Y, L)] = ya / sa
        s_ref[pl.ds(S_Y + L, L)] = yb / sb

        w1k = [vl(OFF_W1 + 16 * k) for k in range(4)]
        b1k = [vl(OFF_B1 + 16 * k) for k in range(4)]

        def h_step(i, carry):
            yv = bcast(s_ref, S_Y + i)
            for k in range(4):
                h_ref[pl.ds(i * 64 + 16 * k, L)] = jnp.maximum(
                    yv * w1k[k] + b1k[k], 0.0)
            return carry

        lax.fori_loop(0, N, h_step, 0)

        vs2 = [u_ref[pl.ds(1264 + 16 * k, L)] for k in range(4)]
        vd2 = [u_ref[pl.ds(1328 + 16 * k, L)] for k in range(4)]

        s_ref[pl.ds(S_AS2 + L, L)] = jnp.full((L,), NEG, jnp.float32)

        def as2_step(i, carry):
            hk = [h_ref[pl.ds(i * 64 + 16 * k, L)] for k in range(4)]
            a_s = hk[0] * vs2[0]
            a_d = hk[0] * vd2[0]
            for k in range(1, 4):
                a_s = a_s + hk[k] * vs2[k]
                a_d = a_d + hk[k] * vd2[k]
            plsc.store_scatter(
                s_ref, [jnp.full((L,), S_AS2 + i, jnp.int32)],
                jnp.broadcast_to(jnp.sum(a_s), (L,)), mask=lane0)
            plsc.store_scatter(
                s_ref, [jnp.full((L,), S_AD2 + i, jnp.int32)],
                jnp.broadcast_to(jnp.sum(a_d), (L,)), mask=lane0)
            return carry

        lax.fori_loop(0, N, as2_step, 0)

        as2a = s_ref[pl.ds(S_AS2, L)]
        as2b = s_ref[pl.ds(S_AS2 + L, L)]

        def l2_step(i, p1v):
            ad2i = bcast(s_ref, S_AD2 + i)
            e_a = _lrelu(as2a + ad2i)
            e_b = _lrelu(as2b + ad2i)
            m = jnp.maximum(jnp.max(e_a), jnp.max(e_b))
            p_a = jnp.exp(e_a - m)
            p_b = jnp.exp(e_b - m)
            rs = 1.0 / jnp.broadcast_to(jnp.sum(p_a) + jnp.sum(p_b), (L,))
            s_ref[pl.ds(S_ALPHA, L)] = p_a
            s_ref[pl.ds(S_ALPHA + L, L)] = p_b

            def m_step(j, mc):
                pj = bcast(s_ref, S_ALPHA + j)
                return tuple(mc[k] + pj * h_ref[pl.ds(j * 64 + 16 * k, L)]
                             for k in range(4))

            mk = lax.fori_loop(0, N, m_step, (zero,) * 4, unroll=2)
            for k in range(4):
                p1v = p1v + mk[k] * rs * u_ref[pl.ds(i * 64 + 16 * k, L)]
            return p1v

        p1v = lax.fori_loop(0, N, l2_step, zero)

        for k in range(8):
            p1v = p1v + u_ref[pl.ds(1392 + 16 * k, L)] * vl(OFF_B2 + 16 * k)

        mmse = vl(OFF_SCAL)[0]
        for k in range(2):
            t = mmse * vl(OFF_WM + 16 * k) + vl(OFF_BM + 16 * k)
            p1v = p1v + t * vl(OFF_WCM + 16 * k)

        z = jnp.sum(p1v) + vl(OFF_BC)[0]
        zv = jnp.broadcast_to(z, (L,))
        out_v[...] = 1.0 / (1.0 + jnp.exp(-zv))
        pltpu.sync_copy(out_v, out_hbm)


@jax.jit
def _run(x, mmse, w1, as1, ad1, b1, w2, as2, ad2, b2, wm, bm, wc, bc):
    mesh = plsc.VectorSubcoreMesh(
        core_axis_name="c", subcore_axis_name="s", num_cores=1,
        num_subcores=16)
    f = pl.kernel(
        _body,
        out_type=jax.ShapeDtypeStruct((L,), jnp.float32),
        mesh=mesh,
        compiler_params=pltpu.CompilerParams(needs_layout_passes=False),
        scratch_types=[
            pltpu.VMEM((TOT,), jnp.float32),
            pltpu.VMEM((N * 64,), jnp.float32),
            pltpu.VMEM((1520,), jnp.float32),
            pltpu.VMEM((128,), jnp.float32),
            pltpu.VMEM((128,), jnp.float32),
            pltpu.VMEM((L,), jnp.float32),
            pltpu.VMEM_SHARED((1520,), jnp.float32),
            pltpu.SemaphoreType.DMA,
        ],
    )
    return f(x, mmse, w1, as1, ad1, b1, w2, as2, ad2, b2, wm, bm, wc, bc)


def kernel(eeg_dem_scores, mmse, W1, a_src1, a_dst1, b1, W2, a_src2,
           a_dst2, b2, Wm, bm, Wc, bc):
    out = _run(eeg_dem_scores.reshape(N), mmse, W1.reshape(64), a_src1,
               a_dst1, b1, W2.reshape(128 * 64), a_src2, a_dst2, b2,
               Wm.reshape(32), bm, Wc.reshape(2464), bc)
    return out[0:1].reshape(1, 1)


# speedup vs baseline: 1.0797x; 1.0244x over previous
import jax
import jax.numpy as jnp
from jax import lax
from jax.experimental import pallas as pl
from jax.experimental.pallas import tpu as pltpu
from jax.experimental.pallas import tpu_sc as plsc

N = 19
L = 16
NEG = -1e30

OFF_X = 0
OFF_W1 = 32
OFF_AS1 = 96
OFF_AD1 = 160
OFF_B1 = 224
OFF_W2 = 288
OFF_AS2 = 8480
OFF_AD2 = 8608
OFF_B2 = 8736
OFF_WCR = 8864
OFF_WCM = 11296
OFF_WM = 11328
OFF_BM = 11360
OFF_SCAL = 11392
OFF_BC = 11408
TOT = 11424

S_AS2 = 0
S_AD2 = 32
S_ALPHA = 64
S_Y = 96
S_ALPHA2 = 128


def _lrelu(t):
    return jnp.where(t >= 0.0, t, 0.2 * t)


def _body(x_hbm, mmse_hbm, w1_hbm, as1_hbm, ad1_hbm, b1_hbm, w2_hbm,
          as2_hbm, ad2_hbm, b2_hbm, wm_hbm, bm_hbm, wc_hbm, bc_hbm,
          out_hbm, buf, h_ref, u_ref, s_ref, ustage, out_v, shared_u, sem):
    tid = lax.axis_index("s")
    lane = lax.iota(jnp.int32, L)
    tail_mask = lane < (N - L)

    copies = [
        (x_hbm, pl.ds(OFF_X, N)),
        (mmse_hbm, pl.ds(OFF_SCAL, 1)),
        (w1_hbm, pl.ds(OFF_W1, 64)),
        (as1_hbm, pl.ds(OFF_AS1, 64)),
        (ad1_hbm, pl.ds(OFF_AD1, 64)),
        (b1_hbm, pl.ds(OFF_B1, 64)),
        (w2_hbm, pl.ds(OFF_W2, 8192)),
        (as2_hbm, pl.ds(OFF_AS2, 128)),
        (ad2_hbm, pl.ds(OFF_AD2, 128)),
        (b2_hbm, pl.ds(OFF_B2, 128)),
        (wm_hbm, pl.ds(OFF_WM, 32)),
        (bm_hbm, pl.ds(OFF_BM, 32)),
        (bc_hbm, pl.ds(OFF_BC, 1)),
    ]
    handles = [pltpu.async_copy(s, buf.at[d], sem) for s, d in copies]
    handles.append(
        pltpu.async_copy(wc_hbm.at[pl.ds(0, N * 128)],
                         buf.at[pl.ds(OFF_WCR, N * 128)], sem))
    handles.append(
        pltpu.async_copy(wc_hbm.at[pl.ds(N * 128, 32)],
                         buf.at[pl.ds(OFF_WCM, 32)], sem))
    for hnd in handles:
        hnd.wait()
    plsc.subcore_barrier()

    def vl(off):
        return buf[pl.ds(off, L)]

    def bcast(ref, idx):
        return plsc.load_gather(ref, [jnp.full((L,), idx, jnp.int32)])

    zero = jnp.zeros((L,), jnp.float32)
    lane0 = lane == 0

    def u_row(i):
        def u_step(c, uc):
            base = OFF_W2 + c * 64
            wc = bcast(buf, OFF_WCR + i * 128 + c)
            return tuple(uc[k] + wc * buf[pl.ds(base + 16 * k, L)]
                         for k in range(4))

        uk = lax.fori_loop(0, 128, u_step, (zero,) * 4, unroll=4)
        for k in range(4):
            ustage[pl.ds(16 * k, L)] = uk[k]
        pltpu.sync_copy(ustage.at[pl.ds(0, 64)],
                        shared_u.at[pl.ds(i * 64, 64)])

    u_row(tid)

    @pl.when(tid < 3)
    def _second_row():
        u_row(tid + L)

    @pl.when(tid == 3)
    def _vsvd_tile():
        def vsvd_step(c, carry):
            base = OFF_W2 + c * 64
            ss = bcast(buf, OFF_AS2 + c)
            sd = bcast(buf, OFF_AD2 + c)
            out = []
            for k in range(4):
                w = buf[pl.ds(base + 16 * k, L)]
                out.append(carry[k] + ss * w)
                out.append(carry[4 + k] + sd * w)
            return tuple(out[0::2]) + tuple(out[1::2])

        vsvd = lax.fori_loop(0, 128, vsvd_step, (zero,) * 8, unroll=2)
        for k in range(8):
            ustage[pl.ds(16 * k, L)] = vsvd[k]
        pltpu.sync_copy(ustage, shared_u.at[pl.ds(1264, 128)])

    @pl.when(tid == 4)
    def _colsum_tile():
        def col_step(i, carry):
            return tuple(
                carry[k] + buf[pl.ds(OFF_WCR + i * 128 + 16 * k, L)]
                for k in range(8))

        cols = lax.fori_loop(0, N, col_step, (zero,) * 8)
        for k in range(8):
            ustage[pl.ds(16 * k, L)] = cols[k]
        pltpu.sync_copy(ustage, shared_u.at[pl.ds(1392, 128)])

    plsc.subcore_barrier()

    @pl.when(tid == 0)
    def _serial():
        pltpu.sync_copy(shared_u, u_ref)

        xa = vl(OFF_X)
        xb = jnp.where(tail_mask, vl(OFF_X + L), 0.0)

        acc_s = vl(OFF_W1) * vl(OFF_AS1)
        acc_d = vl(OFF_W1) * vl(OFF_AD1)
        for k in range(1, 4):
            acc_s = acc_s + vl(OFF_W1 + 16 * k) * vl(OFF_AS1 + 16 * k)
            acc_d = acc_d + vl(OFF_W1 + 16 * k) * vl(OFF_AD1 + 16 * k)
        cs1 = jnp.sum(acc_s)
        cd1 = jnp.sum(acc_d)

        ad1a = xa * cd1
        ad1b = xb * cd1
        negv = jnp.full((L,), NEG, jnp.float32)

        def l1max_step(j, carry):
            mxa, mxb = carry
            asj = bcast(buf, OFF_X + j) * cs1
            return (jnp.maximum(mxa, _lrelu(ad1a + asj)),
                    jnp.maximum(mxb, _lrelu(ad1b + asj)))

        mx1a, mx1b = lax.fori_loop(0, N, l1max_step, (negv, negv))

        def l1sum_step(j, carry):
            sa, sb, ya, yb = carry
            xj = bcast(buf, OFF_X + j)
            asj = xj * cs1
            p_a = jnp.exp(_lrelu(ad1a + asj) - mx1a)
            p_b = jnp.exp(_lrelu(ad1b + asj) - mx1b)
            return (sa + p_a, sb + p_b, ya + p_a * xj, yb + p_b * xj)

        sa, sb, ya, yb = lax.fori_loop(0, N, l1sum_step, (zero,) * 4)
        s_ref[pl.ds(S_Y, L)] = ya / sa
        s_ref[pl.ds(S_Y + L, L)] = yb / sb

        w1k = [vl(OFF_W1 + 16 * k) for k in range(4)]
        b1k = [vl(OFF_B1 + 16 * k) for k in range(4)]

        def h_step(i, carry):
            yv = bcast(s_ref, S_Y + i)
            for k in range(4):
                h_ref[pl.ds(i * 64 + 16 * k, L)] = jnp.maximum(
                    yv * w1k[k] + b1k[k], 0.0)
            return carry

        lax.fori_loop(0, N, h_step, 0)

        vs2 = [u_ref[pl.ds(1264 + 16 * k, L)] for k in range(4)]
        vd2 = [u_ref[pl.ds(1328 + 16 * k, L)] for k in range(4)]

        s_ref[pl.ds(S_AS2 + L, L)] = jnp.full((L,), NEG, jnp.float32)

        def as2_step(i, carry):
            hk = [h_ref[pl.ds(i * 64 + 16 * k, L)] for k in range(4)]
            a_s = hk[0] * vs2[0]
            a_d = hk[0] * vd2[0]
            for k in range(1, 4):
                a_s = a_s + hk[k] * vs2[k]
                a_d = a_d + hk[k] * vd2[k]
            plsc.store_scatter(
                s_ref, [jnp.full((L,), S_AS2 + i, jnp.int32)],
                jnp.broadcast_to(jnp.sum(a_s), (L,)), mask=lane0)
            plsc.store_scatter(
                s_ref, [jnp.full((L,), S_AD2 + i, jnp.int32)],
                jnp.broadcast_to(jnp.sum(a_d), (L,)), mask=lane0)
            return carry

        lax.fori_loop(0, N, as2_step, 0)

        as2a = s_ref[pl.ds(S_AS2, L)]
        as2b = s_ref[pl.ds(S_AS2 + L, L)]

        def row_prep(i, off):
            ad2i = bcast(s_ref, S_AD2 + i)
            e_a = _lrelu(as2a + ad2i)
            e_b = _lrelu(as2b + ad2i)
            m = jnp.maximum(jnp.max(e_a), jnp.max(e_b))
            p_a = jnp.exp(e_a - m)
            p_b = jnp.exp(e_b - m)
            rs = 1.0 / jnp.broadcast_to(jnp.sum(p_a) + jnp.sum(p_b), (L,))
            s_ref[pl.ds(off, L)] = p_a
            s_ref[pl.ds(off + L, L)] = p_b
            return rs

        def l2_pair(i2, p1v):
            i0 = 2 * i2
            i1 = i0 + 1
            rs0 = row_prep(i0, S_ALPHA)
            rs1 = row_prep(i1, S_ALPHA2)

            def m_step(j, mc):
                hk = [h_ref[pl.ds(j * 64 + 16 * k, L)] for k in range(4)]
                pj0 = bcast(s_ref, S_ALPHA + j)
                pj1 = bcast(s_ref, S_ALPHA2 + j)
                return (tuple(mc[k] + pj0 * hk[k] for k in range(4)) +
                        tuple(mc[4 + k] + pj1 * hk[k] for k in range(4)))

            mk = lax.fori_loop(0, N, m_step, (zero,) * 8, unroll=2)
            for k in range(4):
                p1v = p1v + mk[k] * rs0 * u_ref[pl.ds(i0 * 64 + 16 * k, L)]
                p1v = p1v + mk[4 + k] * rs1 * u_ref[
                    pl.ds(i1 * 64 + 16 * k, L)]
            return p1v

        p1v = lax.fori_loop(0, (N - 1) // 2, l2_pair, zero)

        rs18 = row_prep(N - 1, S_ALPHA)

        def m_step18(j, mc):
            pj = bcast(s_ref, S_ALPHA + j)
            return tuple(mc[k] + pj * h_ref[pl.ds(j * 64 + 16 * k, L)]
                         for k in range(4))

        mk18 = lax.fori_loop(0, N, m_step18, (zero,) * 4, unroll=2)
        for k in range(4):
            p1v = p1v + mk18[k] * rs18 * u_ref[
                pl.ds((N - 1) * 64 + 16 * k, L)]

        for k in range(8):
            p1v = p1v + u_ref[pl.ds(1392 + 16 * k, L)] * vl(OFF_B2 + 16 * k)

        mmse = vl(OFF_SCAL)[0]
        for k in range(2):
            t = mmse * vl(OFF_WM + 16 * k) + vl(OFF_BM + 16 * k)
            p1v = p1v + t * vl(OFF_WCM + 16 * k)

        z = jnp.sum(p1v) + vl(OFF_BC)[0]
        zv = jnp.broadcast_to(z, (L,))
        out_v[...] = 1.0 / (1.0 + jnp.exp(-zv))
        pltpu.sync_copy(out_v, out_hbm)


@jax.jit
def _run(x, mmse, w1, as1, ad1, b1, w2, as2, ad2, b2, wm, bm, wc, bc):
    mesh = plsc.VectorSubcoreMesh(
        core_axis_name="c", subcore_axis_name="s", num_cores=1,
        num_subcores=16)
    f = pl.kernel(
        _body,
        out_type=jax.ShapeDtypeStruct((L,), jnp.float32),
        mesh=mesh,
        compiler_params=pltpu.CompilerParams(needs_layout_passes=False),
        scratch_types=[
            pltpu.VMEM((TOT,), jnp.float32),
            pltpu.VMEM((N * 64,), jnp.float32),
            pltpu.VMEM((1520,), jnp.float32),
            pltpu.VMEM((192,), jnp.float32),
            pltpu.VMEM((128,), jnp.float32),
            pltpu.VMEM((L,), jnp.float32),
            pltpu.VMEM_SHARED((1520,), jnp.float32),
            pltpu.SemaphoreType.DMA,
        ],
    )
    return f(x, mmse, w1, as1, ad1, b1, w2, as2, ad2, b2, wm, bm, wc, bc)


def kernel(eeg_dem_scores, mmse, W1, a_src1, a_dst1, b1, W2, a_src2,
           a_dst2, b2, Wm, bm, Wc, bc):
    out = _run(eeg_dem_scores.reshape(N), mmse, W1.reshape(64), a_src1,
               a_dst1, b1, W2.reshape(128 * 64), a_src2, a_dst2, b2,
               Wm.reshape(32), bm, Wc.reshape(2464), bc)
    return out[0:1].reshape(1, 1)
